# Initial kernel scaffold; baseline (speedup 1.0000x reference)
import jax
import jax.numpy as jnp
from jax.experimental import pallas as pl


def kernel(edge_index, emb_weight):
    raise NotImplementedError("write your pallas kernel here")


# speedup vs baseline: 8.0305x; 8.0305x over previous
import functools

import jax
import jax.numpy as jnp
from jax import lax
from jax.experimental import pallas as pl
from jax.experimental.pallas import tpu as pltpu
from jax.experimental.pallas import tpu_sc as plsc

N_NODES = 50000
LATENT = 64
N_EDGES = 800000
N_LAYERS = 3

NC = 2
NS = 16
LANES = 16

HALF = N_NODES // 2
ROWS = 25088
TRASH = HALF
RPT = ROWS // NS

CB = 128
GRP = 8
ICH = CB * GRP
EPT = 50176
E_PAD = EPT * NS
NI = EPT // ICH

_mesh = plsc.VectorSubcoreMesh(core_axis_name="c", subcore_axis_name="s")
_sc_params = pltpu.CompilerParams(use_tc_tiling_on_sc=False)


def _zero_rows(buf, nrows):
    zv = jnp.zeros((LANES,), jnp.float32)

    @pl.loop(0, nrows)
    def _(r):
        @pl.loop(0, LATENT // LANES)
        def _(g):
            buf[r, pl.ds(g * LANES, LANES)] = zv


def _compute_local_idx(tidx, lidx, lo):
    @pl.loop(0, GRP)
    def _(j):
        @pl.loop(0, CB // LANES)
        def _(i):
            t = tidx[j, pl.ds(i * LANES, LANES)]
            inb = (t >= lo) & (t < lo + HALF)
            lidx[j, pl.ds(i * LANES, LANES)] = jnp.where(inb, t - lo, TRASH)


def _drain(acc, out_hbm, sid, cid):
    base = sid * RPT
    out_base = cid * HALF + base
    last_valid = HALF - 15 * RPT

    @pl.when(sid < NS - 1)
    def _():
        pltpu.sync_copy(acc.at[pl.ds(base, RPT)], out_hbm.at[pl.ds(out_base, RPT)])

    @pl.when(sid == NS - 1)
    def _():
        pltpu.sync_copy(
            acc.at[pl.ds(base, last_valid)], out_hbm.at[pl.ds(out_base, last_valid)]
        )


DW = 16


def _sc_degree(to_p):
    @functools.partial(
        pl.kernel,
        out_type=jax.ShapeDtypeStruct((N_NODES, DW), jnp.float32),
        mesh=_mesh,
        scratch_types=[
            pltpu.VMEM_SHARED((ROWS, DW), jnp.float32),
            pltpu.VMEM((GRP, CB), jnp.int32),
            pltpu.VMEM((GRP, CB), jnp.int32),
            pltpu.VMEM((CB, DW), jnp.float32),
        ],
        compiler_params=_sc_params,
    )
    def k(to_hbm, deg_hbm, acc, tidx, lidx, ones):
        cid = lax.axis_index("c")
        sid = lax.axis_index("s")
        lo = cid * HALF

        ones_v = jnp.full((LANES,), 1.0, jnp.float32)
        zv = jnp.zeros((LANES,), jnp.float32)

        @pl.loop(0, CB)
        def _(r):
            ones[r, pl.ds(0, LANES)] = zv

        base = sid * RPT

        @pl.loop(0, RPT // CB)
        def _(m):
            pltpu.sync_copy(ones, acc.at[pl.ds(base + m * CB, CB)])

        rem = RPT - (RPT // CB) * CB
        pltpu.sync_copy(
            ones.at[pl.ds(0, rem)], acc.at[pl.ds(base + (RPT // CB) * CB, rem)]
        )

        @pl.loop(0, CB)
        def _(r):
            ones[r, pl.ds(0, LANES)] = ones_v

        plsc.subcore_barrier()

        rbase = sid * (EPT // CB)

        @pl.loop(0, NI)
        def _(it):
            pltpu.sync_copy(to_hbm.at[pl.ds(rbase + it * GRP, GRP)], tidx)
            _compute_local_idx(tidx, lidx, lo)

            @pl.loop(0, GRP)
            def _(j):
                pltpu.sync_copy(ones, acc.at[lidx.at[j]], add=True)

        plsc.subcore_barrier()
        _drain(acc, deg_hbm, sid, cid)

    return k(to_p)


def _sc_aggregate(from_p, to_p, y):
    @functools.partial(
        pl.kernel,
        out_type=jax.ShapeDtypeStruct((N_NODES, LATENT), jnp.float32),
        mesh=_mesh,
        scratch_types=[
            pltpu.VMEM_SHARED((ROWS, LATENT), jnp.float32),
            pltpu.VMEM((GRP, CB), jnp.int32),
            pltpu.VMEM((GRP, CB), jnp.int32),
            pltpu.VMEM((GRP, CB), jnp.int32),
            pltpu.VMEM((CB, LATENT), jnp.float32),
            pltpu.SemaphoreType.DMA,
        ],
        compiler_params=_sc_params,
    )
    def k(from_hbm, to_hbm, y_hbm, z_hbm, acc, fidx, tidx, lidx, rows, sem):
        cid = lax.axis_index("c")
        sid = lax.axis_index("s")
        lo = cid * HALF

        _zero_rows(rows, CB)
        base = sid * RPT

        @pl.loop(0, RPT // CB)
        def _(m):
            pltpu.sync_copy(rows, acc.at[pl.ds(base + m * CB, CB)])

        rem = RPT - (RPT // CB) * CB
        pltpu.sync_copy(
            rows.at[pl.ds(0, rem)], acc.at[pl.ds(base + (RPT // CB) * CB, rem)]
        )
        plsc.subcore_barrier()

        rbase = sid * (EPT // CB)

        @pl.loop(0, NI)
        def _(it):
            pltpu.sync_copy(from_hbm.at[pl.ds(rbase + it * GRP, GRP)], fidx)
            pltpu.sync_copy(to_hbm.at[pl.ds(rbase + it * GRP, GRP)], tidx)
            _compute_local_idx(tidx, lidx, lo)

            @pl.loop(0, GRP)
            def _(j):
                pltpu.async_copy(y_hbm.at[fidx.at[j]], rows, sem).wait()
                pltpu.sync_copy(rows, acc.at[lidx.at[j]], add=True)

        plsc.subcore_barrier()
        _drain(acc, z_hbm, sid, cid)

    return k(from_p, to_p, y)


def _tc_dinv(deg2d):
    def body(deg_ref, dinv_ref):
        d = deg_ref[...][:, 0:1]
        dinv_ref[...] = jnp.where(d > 0.0, lax.rsqrt(d), 0.0)

    return pl.pallas_call(
        body,
        grid=(N_NODES // _RB,),
        in_specs=[pl.BlockSpec((_RB, DW), lambda i: (i, 0))],
        out_specs=pl.BlockSpec((_RB, 1), lambda i: (i, 0)),
        out_shape=jax.ShapeDtypeStruct((N_NODES, 1), jnp.float32),
    )(deg2d)


_RB = 5000


def _row_spec():
    return pl.BlockSpec((_RB, LATENT), lambda i: (i, 0))


def _d_spec():
    return pl.BlockSpec((_RB, 1), lambda i: (i, 0))


def _tc_scale1(x, dinv2):
    def body(x_ref, d_ref, y_ref):
        y_ref[...] = x_ref[...] * d_ref[...]

    return pl.pallas_call(
        body,
        grid=(N_NODES // _RB,),
        in_specs=[_row_spec(), _d_spec()],
        out_specs=_row_spec(),
        out_shape=jax.ShapeDtypeStruct((N_NODES, LATENT), jnp.float32),
    )(x, dinv2)


def _tc_scale2(z, dinv2):
    def body(z_ref, d_ref, x_ref, y_ref):
        d = d_ref[...]
        zx = z_ref[...] * d
        x_ref[...] = zx
        y_ref[...] = zx * d

    return pl.pallas_call(
        body,
        grid=(N_NODES // _RB,),
        in_specs=[_row_spec(), _d_spec()],
        out_specs=[_row_spec(), _row_spec()],
        out_shape=[
            jax.ShapeDtypeStruct((N_NODES, LATENT), jnp.float32),
            jax.ShapeDtypeStruct((N_NODES, LATENT), jnp.float32),
        ],
    )(z, dinv2)


def _tc_final(emb0, x1, x2, z2, dinv2):
    def body(e_ref, x1_ref, x2_ref, z2_ref, d_ref, o_ref):
        x3 = z2_ref[...] * d_ref[...]
        o_ref[...] = 0.25 * (e_ref[...] + x1_ref[...] + x2_ref[...] + x3)

    return pl.pallas_call(
        body,
        grid=(N_NODES // _RB,),
        in_specs=[_row_spec(), _row_spec(), _row_spec(), _row_spec(), _d_spec()],
        out_specs=_row_spec(),
        out_shape=jax.ShapeDtypeStruct((N_NODES, LATENT), jnp.float32),
    )(emb0, x1, x2, z2, dinv2)


def kernel(edge_index, emb_weight):
    from_ = edge_index[0].astype(jnp.int32)
    to_ = edge_index[1].astype(jnp.int32)
    pad = E_PAD - N_EDGES
    from_p = jnp.concatenate([from_, jnp.zeros((pad,), jnp.int32)]).reshape(
        E_PAD // CB, CB
    )
    to_p = jnp.concatenate([to_, jnp.full((pad,), N_NODES, jnp.int32)]).reshape(
        E_PAD // CB, CB
    )

    deg2d = _sc_degree(to_p)
    dinv2 = _tc_dinv(deg2d)

    y = _tc_scale1(emb_weight, dinv2)
    z0 = _sc_aggregate(from_p, to_p, y)
    x1, y = _tc_scale2(z0, dinv2)
    z1 = _sc_aggregate(from_p, to_p, y)
    x2, y = _tc_scale2(z1, dinv2)
    z2 = _sc_aggregate(from_p, to_p, y)
    out = _tc_final(emb_weight, x1, x2, z2, dinv2)
    return (emb_weight, out)


# speedup vs baseline: 18.4467x; 2.2971x over previous
import functools

import jax
import jax.numpy as jnp
from jax import lax
from jax.experimental import pallas as pl
from jax.experimental.pallas import tpu as pltpu
from jax.experimental.pallas import tpu_sc as plsc

N_NODES = 50000
LATENT = 64
N_EDGES = 800000

NC = 2
NS = 16
LANES = 16
HL = LATENT // 2

ACC_N = 50048
TRASH = N_NODES
RPT = ACC_N // NS
RPT_LAST = N_NODES - (NS - 1) * RPT

CB = 128
RG = 14
NSL = 4
ISL = 3
E_PAD = 802816
EROWS = E_PAD // CB
CPT = EROWS // NS
NR = CPT // RG

DW = 16
CPT_D = EROWS // (NC * NS)
NR_D = CPT_D // RG

ZR = 512

_mesh = plsc.VectorSubcoreMesh(core_axis_name="c", subcore_axis_name="s")
_sc_params = pltpu.CompilerParams(use_tc_tiling_on_sc=False)


def _zero_fill(buf, nrows, width):
    zv = jnp.zeros((LANES,), jnp.float32)

    @pl.loop(0, nrows)
    def _(r):
        for g in range(width // LANES):
            buf[r, pl.ds(g * LANES, LANES)] = zv


def _zero_acc(acc, zbuf, base, width):
    nfull = RPT // ZR
    rem = RPT - nfull * ZR

    @pl.loop(0, nfull)
    def _(m):
        pltpu.sync_copy(zbuf, acc.at[pl.ds(base + m * ZR, ZR)])

    if rem:
        pltpu.sync_copy(zbuf.at[pl.ds(0, rem)], acc.at[pl.ds(base + nfull * ZR, rem)])


def _drain(acc, out_hbm, base, out_base, sid):
    @pl.when(sid < NS - 1)
    def _():
        pltpu.sync_copy(acc.at[pl.ds(base, RPT)], out_hbm.at[pl.ds(out_base, RPT)])

    @pl.when(sid == NS - 1)
    def _():
        pltpu.sync_copy(
            acc.at[pl.ds(base, RPT_LAST)], out_hbm.at[pl.ds(out_base, RPT_LAST)]
        )


def _sc_degree(to_p):
    @functools.partial(
        pl.kernel,
        out_type=jax.ShapeDtypeStruct((NC * N_NODES, DW), jnp.float32),
        mesh=_mesh,
        scratch_types=[
            pltpu.VMEM_SHARED((ACC_N, DW), jnp.float32),
            pltpu.VMEM((2, RG, CB), jnp.int32),
            pltpu.VMEM((CB, DW), jnp.float32),
            pltpu.VMEM((ZR, DW), jnp.float32),
            pltpu.SemaphoreType.DMA,
            pltpu.SemaphoreType.DMA,
            pltpu.SemaphoreType.DMA,
        ],
        compiler_params=_sc_params,
    )
    def k(to_hbm, deg_hbm, acc, tidx, ones, zbuf, isem, ssem0, ssem1):
        cid = lax.axis_index("c")
        sid = lax.axis_index("s")
        ssem = (ssem0, ssem1)
        rbase = cid * (EROWS // NC) + sid * CPT_D

        ones_v = jnp.full((LANES,), 1.0, jnp.float32)

        @pl.loop(0, CB)
        def _(r):
            ones[r, pl.ds(0, LANES)] = ones_v

        _zero_fill(zbuf, ZR, DW)

        pltpu.async_copy(to_hbm.at[pl.ds(rbase, RG)], tidx.at[0], isem)
        _zero_acc(acc, zbuf, sid * RPT, DW)
        plsc.subcore_barrier()

        @pl.loop(0, NR_D // 2)
        def _(i):
            for s in range(2):
                q = i * 2 + s
                pltpu.make_async_copy(
                    to_hbm.at[pl.ds(rbase + q * RG, RG)], tidx.at[s], isem
                ).wait()
                for j in range(RG):
                    pltpu.async_copy(
                        ones, acc.at[tidx.at[s, j]], ssem[s], add=True
                    )
                @pl.when(q > 0)
                def _():
                    for _j in range(RG):
                        pltpu.make_async_copy(
                            ones, acc.at[tidx.at[1 - s, 0]], ssem[1 - s]
                        ).wait()

                @pl.when(q < NR_D - 1)
                def _():
                    pltpu.async_copy(
                        to_hbm.at[pl.ds(rbase + (q + 1) * RG, RG)],
                        tidx.at[1 - s],
                        isem,
                    )

        for _j in range(RG):
            pltpu.make_async_copy(ones, acc.at[tidx.at[1, 0]], ssem[1]).wait()
        plsc.subcore_barrier()
        _drain(acc, deg_hbm, sid * RPT, cid * N_NODES + sid * RPT, sid)

    return k(to_p)


def _sc_aggregate(gfrom_p, to_p, y_flat):
    @functools.partial(
        pl.kernel,
        out_type=jax.ShapeDtypeStruct((NC * N_NODES, HL), jnp.float32),
        mesh=_mesh,
        scratch_types=[
            pltpu.VMEM_SHARED((ACC_N, HL), jnp.float32),
            pltpu.VMEM((ISL, RG, CB), jnp.int32),
            pltpu.VMEM((ISL, RG, CB), jnp.int32),
            pltpu.VMEM((CB, HL), jnp.float32),
            pltpu.VMEM((CB, HL), jnp.float32),
            pltpu.VMEM((CB, HL), jnp.float32),
            pltpu.VMEM((CB, HL), jnp.float32),
            pltpu.SemaphoreType.DMA,
            pltpu.SemaphoreType.DMA,
            pltpu.SemaphoreType.DMA,
            pltpu.SemaphoreType.DMA,
            pltpu.SemaphoreType.DMA,
            pltpu.SemaphoreType.DMA,
            pltpu.SemaphoreType.DMA,
            pltpu.SemaphoreType.DMA,
            pltpu.SemaphoreType.DMA,
        ],
        compiler_params=_sc_params,
    )
    def k(gf_hbm, to_hbm, y_hbm, z_hbm, acc, fidx, tidx, r0, r1, r2, r3,
          isem, g0, g1, g2, g3, s0, s1, s2, s3):
        cid = lax.axis_index("c")
        sid = lax.axis_index("s")
        rows = (r0, r1, r2, r3)
        gsem = (g0, g1, g2, g3)
        ssem = (s0, s1, s2, s3)
        rbase = sid * CPT

        def idx_dma(q, slot):
            pltpu.async_copy(
                gf_hbm.at[cid, pl.ds(rbase + q * RG, RG)], fidx.at[slot], isem
            )
            pltpu.async_copy(
                to_hbm.at[pl.ds(rbase + q * RG, RG)], tidx.at[slot], isem
            )

        def idx_wait(q, slot):
            pltpu.make_async_copy(
                gf_hbm.at[cid, pl.ds(rbase + q * RG, RG)], fidx.at[slot], isem
            ).wait()
            pltpu.make_async_copy(
                to_hbm.at[pl.ds(rbase + q * RG, RG)], tidx.at[slot], isem
            ).wait()

        idx_dma(0, 0)
        idx_dma(1, 1)
        _zero_fill(r0, CB, HL)
        base = sid * RPT
        nfull = RPT // CB
        rem = RPT - nfull * CB

        @pl.loop(0, nfull)
        def _(m):
            pltpu.sync_copy(r0, acc.at[pl.ds(base + m * CB, CB)])

        if rem:
            pltpu.sync_copy(r0.at[pl.ds(0, rem)], acc.at[pl.ds(base + nfull * CB, rem)])
        plsc.subcore_barrier()

        def scatter(slot, j, b):
            pltpu.make_async_copy(rows[b], acc.at[tidx.at[slot, j]], gsem[b]).wait()
            pltpu.async_copy(rows[b], acc.at[tidx.at[slot, j]], ssem[b], add=True)

        @pl.loop(0, NR)
        def _(q):
            slot = lax.rem(q, ISL)
            idx_wait(q, slot)
            for j in range(RG):
                b = j % NSL
                if j < NSL:
                    @pl.when(q > 0)
                    def _():
                        pltpu.make_async_copy(
                            rows[b], acc.at[tidx.at[slot, j]], ssem[b]
                        ).wait()
                else:
                    pltpu.make_async_copy(
                        rows[b], acc.at[tidx.at[slot, j]], ssem[b]
                    ).wait()
                pltpu.async_copy(y_hbm.at[fidx.at[slot, j]], rows[b], gsem[b])
                if j > 0:
                    scatter(slot, j - 1, (j - 1) % NSL)
            scatter(slot, RG - 1, (RG - 1) % NSL)

            @pl.when(q < NR - 2)
            def _():
                idx_dma(q + 2, lax.rem(q + 2, ISL))

        for b in range(NSL):
            pltpu.make_async_copy(rows[b], acc.at[tidx.at[0, 0]], ssem[b]).wait()
        plsc.subcore_barrier()
        _drain(acc, z_hbm, sid * RPT, cid * N_NODES + sid * RPT, sid)

    return k(gfrom_p, to_p, y_flat)


_RB = 5000


def _tc_dinv(deg_st):
    def body(da_ref, db_ref, dinv_ref):
        d = da_ref[...][0, :, 0:1] + db_ref[...][0, :, 0:1]
        dinv_ref[...] = jnp.where(d > 0.0, lax.rsqrt(d), 0.0)

    return pl.pallas_call(
        body,
        grid=(N_NODES // _RB,),
        in_specs=[
            pl.BlockSpec((1, _RB, DW), lambda i: (0, i, 0)),
            pl.BlockSpec((1, _RB, DW), lambda i: (1, i, 0)),
        ],
        out_specs=pl.BlockSpec((_RB, 1), lambda i: (i, 0)),
        out_shape=jax.ShapeDtypeStruct((N_NODES, 1), jnp.float32),
    )(deg_st, deg_st)


def _half_spec():
    return pl.BlockSpec((NC, _RB, HL), lambda i: (0, i, 0))


def _d_spec():
    return pl.BlockSpec((_RB, 1), lambda i: (i, 0))


def _full_spec():
    return pl.BlockSpec((_RB, LATENT), lambda i: (i, 0))


def _tc_scale1(x, dinv2):
    def body(x_ref, d_ref, y_ref):
        d = d_ref[...]
        x = x_ref[...]
        y_ref[...] = jnp.stack([x[:, :HL] * d, x[:, HL:] * d])

    return pl.pallas_call(
        body,
        grid=(N_NODES // _RB,),
        in_specs=[_full_spec(), _d_spec()],
        out_specs=_half_spec(),
        out_shape=jax.ShapeDtypeStruct((NC, N_NODES, HL), jnp.float32),
    )(x, dinv2)


def _tc_scale2(z_st, dinv2):
    def body(z_ref, d_ref, x_ref, y_ref):
        d = d_ref[...][None]
        x = z_ref[...] * d
        x_ref[...] = x
        y_ref[...] = x * d

    return pl.pallas_call(
        body,
        grid=(N_NODES // _RB,),
        in_specs=[_half_spec(), _d_spec()],
        out_specs=[_half_spec(), _half_spec()],
        out_shape=[
            jax.ShapeDtypeStruct((NC, N_NODES, HL), jnp.float32),
            jax.ShapeDtypeStruct((NC, N_NODES, HL), jnp.float32),
        ],
    )(z_st, dinv2)


def _tc_final(emb0, x1_st, x2_st, z2_st, dinv2):
    def body(e_ref, x1_ref, x2_ref, z2_ref, d_ref, o_ref):
        d = d_ref[...][None]
        x3 = z2_ref[...] * d
        s = x1_ref[...] + x2_ref[...] + x3
        both = jnp.concatenate([s[0], s[1]], axis=1)
        o_ref[...] = 0.25 * (e_ref[...] + both)

    return pl.pallas_call(
        body,
        grid=(N_NODES // _RB,),
        in_specs=[_full_spec(), _half_spec(), _half_spec(), _half_spec(), _d_spec()],
        out_specs=_full_spec(),
        out_shape=jax.ShapeDtypeStruct((N_NODES, LATENT), jnp.float32),
    )(emb0, x1_st, x2_st, z2_st, dinv2)


def kernel(edge_index, emb_weight):
    from_ = edge_index[0].astype(jnp.int32)
    to_ = edge_index[1].astype(jnp.int32)
    pad = E_PAD - N_EDGES
    from2d = jnp.concatenate([from_, jnp.zeros((pad,), jnp.int32)]).reshape(
        EROWS, CB
    )
    to_p = jnp.concatenate([to_, jnp.full((pad,), TRASH, jnp.int32)]).reshape(
        EROWS, CB
    )
    gfrom_p = jnp.stack([from2d, from2d + N_NODES])

    deg_st = _sc_degree(to_p).reshape(NC, N_NODES, DW)
    dinv2 = _tc_dinv(deg_st)

    y = _tc_scale1(emb_weight, dinv2)
    z0 = _sc_aggregate(gfrom_p, to_p, y.reshape(NC * N_NODES, HL))
    x1, y = _tc_scale2(z0.reshape(NC, N_NODES, HL), dinv2)
    z1 = _sc_aggregate(gfrom_p, to_p, y.reshape(NC * N_NODES, HL))
    x2, y = _tc_scale2(z1.reshape(NC, N_NODES, HL), dinv2)
    z2 = _sc_aggregate(gfrom_p, to_p, y.reshape(NC * N_NODES, HL))
    out = _tc_final(emb_weight, x1, x2, z2.reshape(NC, N_NODES, HL), dinv2)
    return (emb_weight, out)
